# baseline (speedup 1.0000x reference)
import functools

import jax
import jax.numpy as jnp
from jax import lax
from jax.experimental import pallas as pl
from jax.experimental.pallas import tpu as pltpu
from jax.experimental.pallas import tpu_sc as plsc

B = 16384
D = 512
C = 100000

_INFO = plsc.get_sparse_core_info()
NC = _INFO.num_cores
NS = _INFO.num_subcores
L = _INFO.num_lanes
NW = NC * NS

PW = B // NW
PH = B // NS
HCHUNK = 128
NHC = PH // HCHUNK
HIST = 100352
ZS = HIST // NS
RCH = 64
NRC = PW // RCH
DB = D // L


def _body(feat_hbm, y_hbm, centers_hbm, out_hbm,
          hist_sh, yh_v, idx_v, cnt_v, zbuf, ones_v, rows_v, feat_v,
          acc_buf, sem_r, sem_f):
    c = lax.axis_index("c")
    s = lax.axis_index("s")
    wid = s * NC + c

    for j in range(NHC):
        pltpu.sync_copy(y_hbm.at[pl.ds(s * PH + j * HCHUNK, HCHUNK)],
                        yh_v.at[j])
    pltpu.sync_copy(y_hbm.at[pl.ds(wid * PW, PW)], idx_v)

    zero16 = jnp.zeros((L,), jnp.float32)
    one16 = jnp.ones((L,), jnp.float32)

    def zfill(i, _):
        zbuf[pl.ds(i * L, L)] = zero16
        return 0
    lax.fori_loop(0, ZS // L, zfill, 0)

    def ofill(i, _):
        ones_v[pl.ds(i * L, L)] = one16
        return 0
    lax.fori_loop(0, HCHUNK // L, ofill, 0)

    pltpu.sync_copy(zbuf, hist_sh.at[pl.ds(s * ZS, ZS)])
    plsc.subcore_barrier()

    for j in range(NHC):
        pltpu.sync_copy(ones_v, hist_sh.at[yh_v.at[j]], add=True)
    plsc.subcore_barrier()

    for j in range(PW // HCHUNK):
        pltpu.async_copy(hist_sh.at[idx_v.at[pl.ds(j * HCHUNK, HCHUNK)]],
                         cnt_v.at[pl.ds(j * HCHUNK, HCHUNK)], sem_r).wait()

    def invf(i, _):
        v = cnt_v[pl.ds(i * L, L)]
        cnt_v[pl.ds(i * L, L)] = 1.0 / (v + 1.0)
        return 0
    lax.fori_loop(0, PW // L, invf, 0)

    base = wid * PW
    acc = jnp.zeros((L,), jnp.float32)
    for k in range(NRC):
        pltpu.async_copy(centers_hbm.at[idx_v.at[pl.ds(k * RCH, RCH)]],
                         rows_v, sem_r)
        pltpu.async_copy(feat_hbm.at[pl.ds(base + k * RCH, RCH)],
                         feat_v, sem_f)
        pltpu.make_async_copy(centers_hbm.at[idx_v.at[pl.ds(k * RCH, RCH)]],
                              rows_v, sem_r).wait()
        pltpu.make_async_copy(feat_hbm.at[pl.ds(base + k * RCH, RCH)],
                              feat_v, sem_f).wait()

        def row_body(r, a):
            inv_b = plsc.load_gather(
                cnt_v, [jnp.full((L,), k * RCH + r, jnp.int32)])
            ar = jnp.zeros((L,), jnp.float32)
            for kd in range(DB):
                f = feat_v[r, pl.ds(kd * L, L)]
                cc = rows_v[r, pl.ds(kd * L, L)]
                d = f - cc
                ar = ar + d * d
            return a + ar * inv_b

        acc = lax.fori_loop(0, RCH, row_body, acc)

    acc_buf[...] = acc
    pltpu.sync_copy(acc_buf, out_hbm.at[wid])


@jax.jit
def _sc_center_loss(feat, y, centers):
    mesh = plsc.VectorSubcoreMesh(core_axis_name="c", subcore_axis_name="s")
    run = pl.kernel(
        _body,
        out_type=jax.ShapeDtypeStruct((NW, L), jnp.float32),
        mesh=mesh,
        scratch_types=[
            pltpu.VMEM_SHARED((HIST,), jnp.float32),
            pltpu.VMEM((NHC, HCHUNK), jnp.int32),
            pltpu.VMEM((PW,), jnp.int32),
            pltpu.VMEM((PW,), jnp.float32),
            pltpu.VMEM((ZS,), jnp.float32),
            pltpu.VMEM((HCHUNK,), jnp.float32),
            pltpu.VMEM((RCH, D), jnp.float32),
            pltpu.VMEM((RCH, D), jnp.float32),
            pltpu.VMEM((L,), jnp.float32),
            pltpu.SemaphoreType.DMA,
            pltpu.SemaphoreType.DMA,
        ],
        compiler_params=pltpu.CompilerParams(needs_layout_passes=False),
    )
    return run(feat, y, centers)


def kernel(feat, y, centers):
    partials = _sc_center_loss(feat, y, centers)
    return 0.5 * jnp.sum(partials)


# speedup vs baseline: 1.4918x; 1.4918x over previous
import jax
import jax.numpy as jnp
from jax import lax
from jax.experimental import pallas as pl
from jax.experimental.pallas import tpu as pltpu
from jax.experimental.pallas import tpu_sc as plsc

B = 16384
D = 512
C = 100000

_INFO = plsc.get_sparse_core_info()
NC = _INFO.num_cores
NS = _INFO.num_subcores
L = _INFO.num_lanes
NW = NC * NS

PW = B // NW
PH = B // NS
HCHUNK = 128
NHC = PH // HCHUNK
HIST = 100352
ZS = HIST // NS
CH = 32
NCHK = PW // CH
NBUF = 3
DB = D // L


def _body(feat_hbm, y_hbm, centers_hbm, out_hbm,
          hist_sh, yh_v, idx_v, cnt_v, zbuf, ones_v, rows3, feat3,
          acc_buf, sem_h, sem_r0, sem_r1, sem_r2, sem_f0, sem_f1, sem_f2):
    c = lax.axis_index("c")
    s = lax.axis_index("s")
    wid = s * NC + c
    base = wid * PW
    sem_r = (sem_r0, sem_r1, sem_r2)
    sem_f = (sem_f0, sem_f1, sem_f2)

    def start(g, b):
        pltpu.async_copy(centers_hbm.at[idx_v.at[pl.ds(g * CH, CH)]],
                         rows3.at[b], sem_r[b])
        pltpu.async_copy(feat_hbm.at[pl.ds(base + g * CH, CH)],
                         feat3.at[b], sem_f[b])

    def wait(g, b):
        pltpu.make_async_copy(centers_hbm.at[idx_v.at[pl.ds(g * CH, CH)]],
                              rows3.at[b], sem_r[b]).wait()
        pltpu.make_async_copy(feat_hbm.at[pl.ds(base + g * CH, CH)],
                              feat3.at[b], sem_f[b]).wait()

    pltpu.sync_copy(y_hbm.at[pl.ds(wid * PW, PW)], idx_v)
    start(0, 0)
    start(1, 1)

    for j in range(NHC):
        pltpu.async_copy(y_hbm.at[pl.ds(s * PH + j * HCHUNK, HCHUNK)],
                         yh_v.at[j], sem_h)
    for j in range(NHC):
        pltpu.make_async_copy(y_hbm.at[pl.ds(s * PH + j * HCHUNK, HCHUNK)],
                              yh_v.at[j], sem_h).wait()

    zero16 = jnp.zeros((L,), jnp.float32)
    one16 = jnp.ones((L,), jnp.float32)

    def zfill(i, _):
        for u in range(8):
            zbuf[pl.ds(i * 8 * L + u * L, L)] = zero16
        return 0
    lax.fori_loop(0, ZS // (8 * L), zfill, 0)

    def ofill(i, _):
        ones_v[pl.ds(i * L, L)] = one16
        return 0
    lax.fori_loop(0, HCHUNK // L, ofill, 0)

    pltpu.sync_copy(zbuf, hist_sh.at[pl.ds(s * ZS, ZS)])
    plsc.subcore_barrier()

    for j in range(NHC):
        pltpu.async_copy(ones_v, hist_sh.at[yh_v.at[j]], sem_h, add=True)
    for j in range(NHC):
        pltpu.make_async_copy(ones_v, hist_sh.at[yh_v.at[j]], sem_h).wait()
    plsc.subcore_barrier()

    for j in range(PW // HCHUNK):
        pltpu.async_copy(hist_sh.at[idx_v.at[pl.ds(j * HCHUNK, HCHUNK)]],
                         cnt_v.at[pl.ds(j * HCHUNK, HCHUNK)], sem_h)
    for j in range(PW // HCHUNK):
        pltpu.make_async_copy(
            hist_sh.at[idx_v.at[pl.ds(j * HCHUNK, HCHUNK)]],
            cnt_v.at[pl.ds(j * HCHUNK, HCHUNK)], sem_h).wait()

    def invf(i, _):
        v = cnt_v[pl.ds(i * L, L)]
        cnt_v[pl.ds(i * L, L)] = 1.0 / (v + 1.0)
        return 0
    lax.fori_loop(0, PW // L, invf, 0)

    acc = jnp.zeros((L,), jnp.float32)
    for g in range(NCHK):
        b = g % NBUF
        wait(g, b)
        if g + 2 < NCHK:
            start(g + 2, (g + 2) % NBUF)

        def row_body(r, a, g=g, b=b):
            inv_b = plsc.load_gather(
                cnt_v, [jnp.full((L,), g * CH + r, jnp.int32)])
            accs = [jnp.zeros((L,), jnp.float32) for _ in range(4)]
            for kd in range(DB):
                f = feat3[b, r, pl.ds(kd * L, L)]
                cc = rows3[b, r, pl.ds(kd * L, L)]
                d = f - cc
                accs[kd % 4] = accs[kd % 4] + d * d
            s4 = (accs[0] + accs[1]) + (accs[2] + accs[3])
            return a + s4 * inv_b

        acc = lax.fori_loop(0, CH, row_body, acc)

    acc_buf[...] = acc
    pltpu.sync_copy(acc_buf, out_hbm.at[wid])


@jax.jit
def _sc_center_loss(feat, y, centers):
    mesh = plsc.VectorSubcoreMesh(core_axis_name="c", subcore_axis_name="s")
    run = pl.kernel(
        _body,
        out_type=jax.ShapeDtypeStruct((NW, L), jnp.float32),
        mesh=mesh,
        scratch_types=[
            pltpu.VMEM_SHARED((HIST,), jnp.float32),
            pltpu.VMEM((NHC, HCHUNK), jnp.int32),
            pltpu.VMEM((PW,), jnp.int32),
            pltpu.VMEM((PW,), jnp.float32),
            pltpu.VMEM((ZS,), jnp.float32),
            pltpu.VMEM((HCHUNK,), jnp.float32),
            pltpu.VMEM((NBUF, CH, D), jnp.float32),
            pltpu.VMEM((NBUF, CH, D), jnp.float32),
            pltpu.VMEM((L,), jnp.float32),
            pltpu.SemaphoreType.DMA,
            pltpu.SemaphoreType.DMA,
            pltpu.SemaphoreType.DMA,
            pltpu.SemaphoreType.DMA,
            pltpu.SemaphoreType.DMA,
            pltpu.SemaphoreType.DMA,
            pltpu.SemaphoreType.DMA,
        ],
        compiler_params=pltpu.CompilerParams(needs_layout_passes=False),
    )
    return run(feat, y, centers)


def kernel(feat, y, centers):
    partials = _sc_center_loss(feat, y, centers)
    return 0.5 * jnp.sum(partials)


# speedup vs baseline: 1.6257x; 1.0898x over previous
import jax
import jax.numpy as jnp
from jax import lax
from jax.experimental import pallas as pl
from jax.experimental.pallas import tpu as pltpu
from jax.experimental.pallas import tpu_sc as plsc

B = 16384
D = 512
C = 100000

_INFO = plsc.get_sparse_core_info()
NC = _INFO.num_cores
NS = _INFO.num_subcores
L = _INFO.num_lanes
NW = NC * NS

PW = B // NW
PH = B // NS
HCHUNK = 128
NHC = PH // HCHUNK
HIST = 100352
ZS = HIST // NS
CH = 32
NCHK = PW // CH
NBUF = 3
DB = D // L


def _body(feat_hbm, y_hbm, centers_hbm, out_hbm,
          hist_sh, yh_v, idx_v, cnt_v, zbuf, ones_v, rows3, feat3,
          acc_buf, sem_h, sem_r0, sem_r1, sem_r2, sem_f0, sem_f1, sem_f2):
    c = lax.axis_index("c")
    s = lax.axis_index("s")
    wid = s * NC + c
    base = wid * PW
    sem_r = (sem_r0, sem_r1, sem_r2)
    sem_f = (sem_f0, sem_f1, sem_f2)

    def start(g, b):
        pltpu.async_copy(centers_hbm.at[idx_v.at[pl.ds(g * CH, CH)]],
                         rows3.at[b], sem_r[b])
        pltpu.async_copy(feat_hbm.at[pl.ds(base + g * CH, CH)],
                         feat3.at[b], sem_f[b])

    def wait(g, b):
        pltpu.make_async_copy(centers_hbm.at[idx_v.at[pl.ds(g * CH, CH)]],
                              rows3.at[b], sem_r[b]).wait()
        pltpu.make_async_copy(feat_hbm.at[pl.ds(base + g * CH, CH)],
                              feat3.at[b], sem_f[b]).wait()

    pltpu.sync_copy(y_hbm.at[pl.ds(wid * PW, PW)], idx_v)
    start(0, 0)
    start(1, 1)

    for j in range(NHC):
        pltpu.async_copy(y_hbm.at[pl.ds(s * PH + j * HCHUNK, HCHUNK)],
                         yh_v.at[j], sem_h)
    for j in range(NHC):
        pltpu.make_async_copy(y_hbm.at[pl.ds(s * PH + j * HCHUNK, HCHUNK)],
                              yh_v.at[j], sem_h).wait()

    zero16 = jnp.zeros((L,), jnp.float32)
    one16 = jnp.ones((L,), jnp.float32)

    def zfill(i, _):
        for u in range(8):
            zbuf[pl.ds(i * 8 * L + u * L, L)] = zero16
        return 0
    lax.fori_loop(0, ZS // (8 * L), zfill, 0)

    def ofill(i, _):
        ones_v[pl.ds(i * L, L)] = one16
        return 0
    lax.fori_loop(0, HCHUNK // L, ofill, 0)

    pltpu.sync_copy(zbuf, hist_sh.at[pl.ds(s * ZS, ZS)])
    plsc.subcore_barrier()

    for j in range(NHC):
        pltpu.async_copy(ones_v, hist_sh.at[yh_v.at[j]], sem_h, add=True)
    for j in range(NHC):
        pltpu.make_async_copy(ones_v, hist_sh.at[yh_v.at[j]], sem_h).wait()
    plsc.subcore_barrier()

    for j in range(PW // HCHUNK):
        pltpu.async_copy(hist_sh.at[idx_v.at[pl.ds(j * HCHUNK, HCHUNK)]],
                         cnt_v.at[pl.ds(j * HCHUNK, HCHUNK)], sem_h)
    for j in range(PW // HCHUNK):
        pltpu.make_async_copy(
            hist_sh.at[idx_v.at[pl.ds(j * HCHUNK, HCHUNK)]],
            cnt_v.at[pl.ds(j * HCHUNK, HCHUNK)], sem_h).wait()

    def invf(i, _):
        v = cnt_v[pl.ds(i * L, L)]
        cnt_v[pl.ds(i * L, L)] = 1.0 / (v + 1.0)
        return 0
    lax.fori_loop(0, PW // L, invf, 0)

    def chunk_compute(g, b, a):
        def row_body(r, a2):
            inv_b = plsc.load_gather(
                cnt_v, [jnp.full((L,), g * CH + r, jnp.int32)])
            accs = [jnp.zeros((L,), jnp.float32) for _ in range(4)]
            for kd in range(DB):
                f = feat3[b, r, pl.ds(kd * L, L)]
                cc = rows3[b, r, pl.ds(kd * L, L)]
                d = f - cc
                accs[kd % 4] = accs[kd % 4] + d * d
            s4 = (accs[0] + accs[1]) + (accs[2] + accs[3])
            return a2 + s4 * inv_b

        return lax.fori_loop(0, CH, row_body, a)

    def round_body(t, a):
        for j in range(NBUF):
            g = NBUF * t + j
            wait(g, j)
            nxt = g + 2

            @pl.when(nxt < NCHK)
            def _(nxt=nxt, j=j):
                start(nxt, (j + 2) % NBUF)

            a = chunk_compute(g, j, a)
        return a

    acc = lax.fori_loop(0, NCHK // NBUF, round_body,
                        jnp.zeros((L,), jnp.float32))
    for g in range(NBUF * (NCHK // NBUF), NCHK):
        wait(g, g % NBUF)
        acc = chunk_compute(g, g % NBUF, acc)

    acc_buf[...] = acc
    pltpu.sync_copy(acc_buf, out_hbm.at[wid])


@jax.jit
def _sc_center_loss(feat, y, centers):
    mesh = plsc.VectorSubcoreMesh(core_axis_name="c", subcore_axis_name="s")
    run = pl.kernel(
        _body,
        out_type=jax.ShapeDtypeStruct((NW, L), jnp.float32),
        mesh=mesh,
        scratch_types=[
            pltpu.VMEM_SHARED((HIST,), jnp.float32),
            pltpu.VMEM((NHC, HCHUNK), jnp.int32),
            pltpu.VMEM((PW,), jnp.int32),
            pltpu.VMEM((PW,), jnp.float32),
            pltpu.VMEM((ZS,), jnp.float32),
            pltpu.VMEM((HCHUNK,), jnp.float32),
            pltpu.VMEM((NBUF, CH, D), jnp.float32),
            pltpu.VMEM((NBUF, CH, D), jnp.float32),
            pltpu.VMEM((L,), jnp.float32),
            pltpu.SemaphoreType.DMA,
            pltpu.SemaphoreType.DMA,
            pltpu.SemaphoreType.DMA,
            pltpu.SemaphoreType.DMA,
            pltpu.SemaphoreType.DMA,
            pltpu.SemaphoreType.DMA,
            pltpu.SemaphoreType.DMA,
        ],
        compiler_params=pltpu.CompilerParams(needs_layout_passes=False),
    )
    return run(feat, y, centers)


def kernel(feat, y, centers):
    partials = _sc_center_loss(feat, y, centers)
    return 0.5 * jnp.sum(partials)


# speedup vs baseline: 1.7409x; 1.0709x over previous
import jax
import jax.numpy as jnp
from jax import lax
from jax.experimental import pallas as pl
from jax.experimental.pallas import tpu as pltpu
from jax.experimental.pallas import tpu_sc as plsc

B = 16384
D = 512
C = 100000

_INFO = plsc.get_sparse_core_info()
NC = _INFO.num_cores
NS = _INFO.num_subcores
L = _INFO.num_lanes
NW = NC * NS

PW = B // NW
PH = B // NS
HCHUNK = 128
NHC = PH // HCHUNK
HIST = 100352
ZS = HIST // NS
CH = 32
NCHK = PW // CH
NBUF = 3
DB = D // L


def _body(feat_hbm, y_hbm, centers_hbm, out_hbm,
          hist_sh, yh_v, idx_v, cnt_v, zbuf, ones_v, rows3, feat3,
          acc_buf, sem_h, sem_r0, sem_f0):
    c = lax.axis_index("c")
    s = lax.axis_index("s")
    wid = s * NC + c
    base = wid * PW
    def start(g, b):
        pltpu.async_copy(centers_hbm.at[idx_v.at[pl.ds(g * CH, CH)]],
                         rows3.at[b], sem_r0)
        pltpu.async_copy(feat_hbm.at[pl.ds(base + g * CH, CH)],
                         feat3.at[b], sem_f0)

    def wait(g, b):
        pltpu.make_async_copy(centers_hbm.at[idx_v.at[pl.ds(g * CH, CH)]],
                              rows3.at[b], sem_r0).wait()
        pltpu.make_async_copy(feat_hbm.at[pl.ds(base + g * CH, CH)],
                              feat3.at[b], sem_f0).wait()

    pltpu.sync_copy(y_hbm.at[pl.ds(wid * PW, PW)], idx_v)
    start(0, 0)
    start(1, 1)

    for j in range(NHC):
        pltpu.async_copy(y_hbm.at[pl.ds(s * PH + j * HCHUNK, HCHUNK)],
                         yh_v.at[j], sem_h)
    for j in range(NHC):
        pltpu.make_async_copy(y_hbm.at[pl.ds(s * PH + j * HCHUNK, HCHUNK)],
                              yh_v.at[j], sem_h).wait()

    zero16 = jnp.zeros((L,), jnp.float32)
    one16 = jnp.ones((L,), jnp.float32)

    def zfill(i, _):
        for u in range(8):
            zbuf[pl.ds(i * 8 * L + u * L, L)] = zero16
        return 0
    lax.fori_loop(0, ZS // (8 * L), zfill, 0)

    def ofill(i, _):
        ones_v[pl.ds(i * L, L)] = one16
        return 0
    lax.fori_loop(0, HCHUNK // L, ofill, 0)

    pltpu.sync_copy(zbuf, hist_sh.at[pl.ds(s * ZS, ZS)])
    plsc.subcore_barrier()

    for j in range(NHC):
        pltpu.async_copy(ones_v, hist_sh.at[yh_v.at[j]], sem_h, add=True)
    for j in range(NHC):
        pltpu.make_async_copy(ones_v, hist_sh.at[yh_v.at[j]], sem_h).wait()
    plsc.subcore_barrier()

    for j in range(PW // HCHUNK):
        pltpu.async_copy(hist_sh.at[idx_v.at[pl.ds(j * HCHUNK, HCHUNK)]],
                         cnt_v.at[pl.ds(j * HCHUNK, HCHUNK)], sem_h)
    for j in range(PW // HCHUNK):
        pltpu.make_async_copy(
            hist_sh.at[idx_v.at[pl.ds(j * HCHUNK, HCHUNK)]],
            cnt_v.at[pl.ds(j * HCHUNK, HCHUNK)], sem_h).wait()

    def invf(i, _):
        v = cnt_v[pl.ds(i * L, L)]
        cnt_v[pl.ds(i * L, L)] = 1.0 / (v + 1.0)
        return 0
    lax.fori_loop(0, PW // L, invf, 0)

    def chunk_compute(g, b, a):
        def row_body(r, a2):
            inv_b = plsc.load_gather(
                cnt_v, [jnp.full((L,), g * CH + r, jnp.int32)])
            accs = [jnp.zeros((L,), jnp.float32) for _ in range(4)]
            for kd in range(DB):
                f = feat3[b, r, pl.ds(kd * L, L)]
                cc = rows3[b, r, pl.ds(kd * L, L)]
                d = f - cc
                accs[kd % 4] = accs[kd % 4] + d * d
            s4 = (accs[0] + accs[1]) + (accs[2] + accs[3])
            return a2 + s4 * inv_b

        return lax.fori_loop(0, CH, row_body, a)

    def round_body(g, a):
        b = lax.rem(g, NBUF)
        wait(g, b)
        nxt = g + 2

        @pl.when(nxt < NCHK)
        def _():
            start(nxt, lax.rem(nxt, NBUF))

        return chunk_compute(g, b, a)

    acc = lax.fori_loop(0, NCHK, round_body, jnp.zeros((L,), jnp.float32))

    acc_buf[...] = acc
    pltpu.sync_copy(acc_buf, out_hbm.at[wid])


@jax.jit
def _sc_center_loss(feat, y, centers):
    mesh = plsc.VectorSubcoreMesh(core_axis_name="c", subcore_axis_name="s")
    run = pl.kernel(
        _body,
        out_type=jax.ShapeDtypeStruct((NW, L), jnp.float32),
        mesh=mesh,
        scratch_types=[
            pltpu.VMEM_SHARED((HIST,), jnp.float32),
            pltpu.VMEM((NHC, HCHUNK), jnp.int32),
            pltpu.VMEM((PW,), jnp.int32),
            pltpu.VMEM((PW,), jnp.float32),
            pltpu.VMEM((ZS,), jnp.float32),
            pltpu.VMEM((HCHUNK,), jnp.float32),
            pltpu.VMEM((NBUF, CH, D), jnp.float32),
            pltpu.VMEM((NBUF, CH, D), jnp.float32),
            pltpu.VMEM((L,), jnp.float32),
            pltpu.SemaphoreType.DMA,
            pltpu.SemaphoreType.DMA,
            pltpu.SemaphoreType.DMA,
        ],
        compiler_params=pltpu.CompilerParams(needs_layout_passes=False),
    )
    return run(feat, y, centers)


def kernel(feat, y, centers):
    partials = _sc_center_loss(feat, y, centers)
    return 0.5 * jnp.sum(partials)


# speedup vs baseline: 1.7425x; 1.0009x over previous
import jax
import jax.numpy as jnp
from jax import lax
from jax.experimental import pallas as pl
from jax.experimental.pallas import tpu as pltpu
from jax.experimental.pallas import tpu_sc as plsc

B = 16384
D = 512
C = 100000

_INFO = plsc.get_sparse_core_info()
NC = _INFO.num_cores
NS = _INFO.num_subcores
L = _INFO.num_lanes
NW = NC * NS

PW = B // NW
PH = B // NS
HCHUNK = 128
NHC = PH // HCHUNK
HIST = 100352
ZS = HIST // NS
CH = 32
NCHK = PW // CH
NBUF = 3
DB = D // L


def _body(feat_hbm, y_hbm, centers_hbm, out_hbm,
          hist_sh, yh_v, idx_v, cnt_v, zbuf, ones_v, rows3, feat3,
          acc_buf, sem_h, sem_r0, sem_f0):
    c = lax.axis_index("c")
    s = lax.axis_index("s")
    wid = s * NC + c
    base = wid * PW
    def start(g, b):
        pltpu.async_copy(centers_hbm.at[idx_v.at[pl.ds(g * CH, CH)]],
                         rows3.at[b], sem_r0)
        pltpu.async_copy(feat_hbm.at[pl.ds(base + g * CH, CH)],
                         feat3.at[b], sem_f0)

    def wait(g, b):
        pltpu.make_async_copy(centers_hbm.at[idx_v.at[pl.ds(g * CH, CH)]],
                              rows3.at[b], sem_r0).wait()
        pltpu.make_async_copy(feat_hbm.at[pl.ds(base + g * CH, CH)],
                              feat3.at[b], sem_f0).wait()

    pltpu.sync_copy(y_hbm.at[pl.ds(wid * PW, PW)], idx_v)
    start(0, 0)
    start(1, 1)

    for j in range(NHC):
        pltpu.async_copy(y_hbm.at[pl.ds(s * PH + j * HCHUNK, HCHUNK)],
                         yh_v.at[j], sem_h)
    for j in range(NHC):
        pltpu.make_async_copy(y_hbm.at[pl.ds(s * PH + j * HCHUNK, HCHUNK)],
                              yh_v.at[j], sem_h).wait()

    zero16 = jnp.zeros((L,), jnp.float32)
    one16 = jnp.ones((L,), jnp.float32)

    def zfill(i, _):
        for u in range(8):
            zbuf[pl.ds(i * 8 * L + u * L, L)] = zero16
        return 0
    lax.fori_loop(0, ZS // (8 * L), zfill, 0)

    def ofill(i, _):
        ones_v[pl.ds(i * L, L)] = one16
        return 0
    lax.fori_loop(0, HCHUNK // L, ofill, 0)

    pltpu.sync_copy(zbuf, hist_sh.at[pl.ds(s * ZS, ZS)])
    plsc.subcore_barrier()

    for j in range(NHC):
        pltpu.async_copy(ones_v, hist_sh.at[yh_v.at[j]], sem_h, add=True)
    for j in range(NHC):
        pltpu.make_async_copy(ones_v, hist_sh.at[yh_v.at[j]], sem_h).wait()
    plsc.subcore_barrier()

    for j in range(PW // HCHUNK):
        pltpu.async_copy(hist_sh.at[idx_v.at[pl.ds(j * HCHUNK, HCHUNK)]],
                         cnt_v.at[pl.ds(j * HCHUNK, HCHUNK)], sem_h)
    for j in range(PW // HCHUNK):
        pltpu.make_async_copy(
            hist_sh.at[idx_v.at[pl.ds(j * HCHUNK, HCHUNK)]],
            cnt_v.at[pl.ds(j * HCHUNK, HCHUNK)], sem_h).wait()

    def invf(i, _):
        v = cnt_v[pl.ds(i * L, L)]
        cnt_v[pl.ds(i * L, L)] = 1.0 / (v + 1.0)
        return 0
    lax.fori_loop(0, PW // L, invf, 0)

    def chunk_compute(g, b, a):
        def group_body(q, a1):
            inv16 = cnt_v[pl.ds(g * CH + q * L, L)]

            def row_body(r2, a2):
                inv_b = inv16.at[jnp.full((L,), r2, jnp.int32)].get(
                    mode="promise_in_bounds")
                r = q * L + r2
                accs = [jnp.zeros((L,), jnp.float32) for _ in range(4)]
                for kd in range(DB):
                    f = feat3[b, r, pl.ds(kd * L, L)]
                    cc = rows3[b, r, pl.ds(kd * L, L)]
                    d = f - cc
                    accs[kd % 4] = accs[kd % 4] + d * d
                s4 = (accs[0] + accs[1]) + (accs[2] + accs[3])
                return a2 + s4 * inv_b

            return lax.fori_loop(0, L, row_body, a1)

        return lax.fori_loop(0, CH // L, group_body, a)

    def round_body(g, a):
        b = lax.rem(g, NBUF)
        wait(g, b)
        nxt = g + 2

        @pl.when(nxt < NCHK)
        def _():
            start(nxt, lax.rem(nxt, NBUF))

        return chunk_compute(g, b, a)

    acc = lax.fori_loop(0, NCHK, round_body, jnp.zeros((L,), jnp.float32))

    acc_buf[...] = acc
    pltpu.sync_copy(acc_buf, out_hbm.at[wid])


@jax.jit
def _sc_center_loss(feat, y, centers):
    mesh = plsc.VectorSubcoreMesh(core_axis_name="c", subcore_axis_name="s")
    run = pl.kernel(
        _body,
        out_type=jax.ShapeDtypeStruct((NW, L), jnp.float32),
        mesh=mesh,
        scratch_types=[
            pltpu.VMEM_SHARED((HIST,), jnp.float32),
            pltpu.VMEM((NHC, HCHUNK), jnp.int32),
            pltpu.VMEM((PW,), jnp.int32),
            pltpu.VMEM((PW,), jnp.float32),
            pltpu.VMEM((ZS,), jnp.float32),
            pltpu.VMEM((HCHUNK,), jnp.float32),
            pltpu.VMEM((NBUF, CH, D), jnp.float32),
            pltpu.VMEM((NBUF, CH, D), jnp.float32),
            pltpu.VMEM((L,), jnp.float32),
            pltpu.SemaphoreType.DMA,
            pltpu.SemaphoreType.DMA,
            pltpu.SemaphoreType.DMA,
        ],
        compiler_params=pltpu.CompilerParams(needs_layout_passes=False),
    )
    return run(feat, y, centers)


def kernel(feat, y, centers):
    partials = _sc_center_loss(feat, y, centers)
    return 0.5 * jnp.sum(partials)
